# Initial kernel scaffold; baseline (speedup 1.0000x reference)
import jax
import jax.numpy as jnp
from jax.experimental import pallas as pl


def kernel(x, W_enc, b_enc, W_dec, b_dec):
    raise NotImplementedError("write your pallas kernel here")


# speedup vs baseline: 14.2439x; 14.2439x over previous
import functools

import jax
import jax.numpy as jnp
from jax.experimental import pallas as pl

K = 128


def _enc_kernel(x_ref, w_ref, b_ref, pre_ref):
    acc = jax.lax.dot_general(
        x_ref[...], w_ref[...],
        dimension_numbers=(((1,), (0,)), ((), ())),
        preferred_element_type=jnp.float32,
    )
    pre_ref[...] = jnp.maximum(acc + b_ref[...], 0.0)


def _encode(x, w, b, *, block_m=256, block_n=512):
    m, d = x.shape
    n = w.shape[1]
    grid = (n // block_n, m // block_m)
    return pl.pallas_call(
        _enc_kernel,
        grid=grid,
        in_specs=[
            pl.BlockSpec((block_m, d), lambda j, i: (i, 0)),
            pl.BlockSpec((d, block_n), lambda j, i: (0, j)),
            pl.BlockSpec((block_n,), lambda j, i: (j,)),
        ],
        out_specs=pl.BlockSpec((block_m, block_n), lambda j, i: (i, j)),
        out_shape=jax.ShapeDtypeStruct((m, n), jnp.float32),
    )(x, w, b)


def _topk_mask_kernel(pre_ref, z_ref, zb_ref):
    y = pre_ref[...]
    yi = jax.lax.bitcast_convert_type(y, jnp.int32)

    def body(it, t):
        cand = t | (1 << (30 - it))
        cnt = jnp.sum((yi >= cand).astype(jnp.int32), axis=1, keepdims=True)
        return jnp.where(cnt >= K, cand, t)

    t = jax.lax.fori_loop(0, 31, body, jnp.zeros((y.shape[0], 1), jnp.int32))
    z = jnp.where(yi >= t, y, 0.0)
    z_ref[...] = z
    zb_ref[...] = z.astype(jnp.bfloat16)


def _topk_mask(pre, *, block_m=128):
    m, n = pre.shape
    return pl.pallas_call(
        _topk_mask_kernel,
        grid=(m // block_m,),
        in_specs=[pl.BlockSpec((block_m, n), lambda i: (i, 0))],
        out_specs=[
            pl.BlockSpec((block_m, n), lambda i: (i, 0)),
            pl.BlockSpec((block_m, n), lambda i: (i, 0)),
        ],
        out_shape=[
            jax.ShapeDtypeStruct((m, n), jnp.float32),
            jax.ShapeDtypeStruct((m, n), jnp.bfloat16),
        ],
    )(pre)


def _dec_kernel(z_ref, w_ref, b_ref, out_ref):
    acc = jax.lax.dot_general(
        z_ref[...], w_ref[...],
        dimension_numbers=(((1,), (0,)), ((), ())),
        preferred_element_type=jnp.float32,
    )
    out_ref[...] = acc + b_ref[...]


def _decode(zb, w, b, *, block_m=256, block_n=512):
    m, h = zb.shape
    n = w.shape[1]
    grid = (n // block_n, m // block_m)
    return pl.pallas_call(
        _dec_kernel,
        grid=grid,
        in_specs=[
            pl.BlockSpec((block_m, h), lambda j, i: (i, 0)),
            pl.BlockSpec((h, block_n), lambda j, i: (0, j)),
            pl.BlockSpec((block_n,), lambda j, i: (j,)),
        ],
        out_specs=pl.BlockSpec((block_m, block_n), lambda j, i: (i, j)),
        out_shape=jax.ShapeDtypeStruct((m, n), jnp.float32),
    )(zb, w, b)


def kernel(x, W_enc, b_enc, W_dec, b_dec):
    pre = _encode(x, W_enc, b_enc)
    z, zb = _topk_mask(pre)
    x_recon = _decode(zb, W_dec.astype(jnp.bfloat16), b_dec)
    return (z, x_recon)


# speedup vs baseline: 16.4243x; 1.1531x over previous
import functools

import jax
import jax.numpy as jnp
from jax.experimental import pallas as pl

K = 128


def _enc_kernel(x_ref, w_ref, b_ref, pre_ref):
    acc = jax.lax.dot_general(
        x_ref[...], w_ref[...],
        dimension_numbers=(((1,), (0,)), ((), ())),
        preferred_element_type=jnp.float32,
    )
    pre_ref[...] = jnp.maximum(acc + b_ref[...], 0.0)


def _encode(x, w, b, *, block_m=512, block_n=1024):
    m, d = x.shape
    n = w.shape[1]
    grid = (n // block_n, m // block_m)
    return pl.pallas_call(
        _enc_kernel,
        grid=grid,
        in_specs=[
            pl.BlockSpec((block_m, d), lambda j, i: (i, 0)),
            pl.BlockSpec((d, block_n), lambda j, i: (0, j)),
            pl.BlockSpec((block_n,), lambda j, i: (j,)),
        ],
        out_specs=pl.BlockSpec((block_m, block_n), lambda j, i: (i, j)),
        out_shape=jax.ShapeDtypeStruct((m, n), jnp.float32),
    )(x, w, b)


def _topk_mask_kernel(pre_ref, z_ref, zb_ref):
    y = pre_ref[...]
    yi = jax.lax.bitcast_convert_type(y, jnp.int32)

    def body(it, t):
        cand = t | (1 << (30 - it))
        cnt = jnp.sum((yi >= cand).astype(jnp.int32), axis=1, keepdims=True)
        return jnp.where(cnt >= K, cand, t)

    t = jax.lax.fori_loop(0, 31, body, jnp.zeros((y.shape[0], 1), jnp.int32))
    z = jnp.where(yi >= t, y, 0.0)
    z_ref[...] = z
    zb_ref[...] = z.astype(jnp.bfloat16)


def _topk_mask(pre, *, block_m=128):
    m, n = pre.shape
    return pl.pallas_call(
        _topk_mask_kernel,
        grid=(m // block_m,),
        in_specs=[pl.BlockSpec((block_m, n), lambda i: (i, 0))],
        out_specs=[
            pl.BlockSpec((block_m, n), lambda i: (i, 0)),
            pl.BlockSpec((block_m, n), lambda i: (i, 0)),
        ],
        out_shape=[
            jax.ShapeDtypeStruct((m, n), jnp.float32),
            jax.ShapeDtypeStruct((m, n), jnp.bfloat16),
        ],
    )(pre)


def _dec_kernel(z_ref, w_ref, b_ref, out_ref):
    acc = jax.lax.dot_general(
        z_ref[...], w_ref[...],
        dimension_numbers=(((1,), (0,)), ((), ())),
        preferred_element_type=jnp.float32,
    )
    out_ref[...] = acc + b_ref[...]


def _decode(zb, w, b, *, block_m=256, block_n=512):
    m, h = zb.shape
    n = w.shape[1]
    grid = (n // block_n, m // block_m)
    return pl.pallas_call(
        _dec_kernel,
        grid=grid,
        in_specs=[
            pl.BlockSpec((block_m, h), lambda j, i: (i, 0)),
            pl.BlockSpec((h, block_n), lambda j, i: (0, j)),
            pl.BlockSpec((block_n,), lambda j, i: (j,)),
        ],
        out_specs=pl.BlockSpec((block_m, block_n), lambda j, i: (i, j)),
        out_shape=jax.ShapeDtypeStruct((m, n), jnp.float32),
    )(zb, w, b)


def kernel(x, W_enc, b_enc, W_dec, b_dec):
    pre = _encode(x.astype(jnp.bfloat16), W_enc.astype(jnp.bfloat16), b_enc)
    z, zb = _topk_mask(pre)
    x_recon = _decode(zb, W_dec.astype(jnp.bfloat16), b_dec)
    return (z, x_recon)
